# baseline (speedup 1.0000x reference)
import functools

import jax
import jax.numpy as jnp
from jax import lax
from jax.experimental import pallas as pl
from jax.experimental.pallas import tpu as pltpu
from jax.experimental.pallas import tpu_sc as plsc

_NUM_WORKERS = 32
_CHUNK = 128


@functools.partial(jax.jit, static_argnums=(3,))
def _sc_embed_concat(x2, labels2, table, n_tokens):
    d_x = x2.shape[1]
    d_e = table.shape[1]
    d_out = d_x + d_e
    per_w = n_tokens // _NUM_WORKERS
    steps = per_w // _CHUNK
    mesh = plsc.VectorSubcoreMesh(core_axis_name="c", subcore_axis_name="s")

    @functools.partial(
        pl.kernel,
        mesh=mesh,
        out_type=jax.ShapeDtypeStruct((n_tokens, d_out), jnp.float32),
        scratch_types=[
            pltpu.VMEM((steps, _CHUNK), jnp.int32),
            pltpu.VMEM((_CHUNK, d_e), jnp.float32),
            pltpu.VMEM((_CHUNK, d_e), jnp.float32),
            pltpu.SemaphoreType.DMA,
            pltpu.SemaphoreType.DMA,
            pltpu.SemaphoreType.DMA,
        ],
    )
    def k(x_hbm, lab_hbm, tab_hbm, out_hbm, idx_v, emb0, emb1, sem_x, sem0, sem1):
        wid = lax.axis_index("s") * 2 + lax.axis_index("c")
        base = wid * per_w
        rows = pl.ds(base, per_w)

        x_copy = pltpu.async_copy(x_hbm.at[rows, :], out_hbm.at[rows, pl.ds(0, d_x)], sem_x)
        pltpu.sync_copy(lab_hbm.at[pl.ds(wid * steps, steps), :], idx_v)

        embs = (emb0, emb1)
        sems = (sem0, sem1)
        copies = [None, None]
        copies[0] = pltpu.async_copy(tab_hbm.at[idx_v.at[0]], emb0, sem0)
        for j in range(steps):
            nxt = j + 1
            if nxt < steps:
                copies[nxt % 2] = pltpu.async_copy(
                    tab_hbm.at[idx_v.at[nxt]], embs[nxt % 2], sems[nxt % 2])
            copies[j % 2].wait()
            pltpu.sync_copy(
                embs[j % 2],
                out_hbm.at[pl.ds(base + j * _CHUNK, _CHUNK), pl.ds(d_x, d_e)])
        x_copy.wait()

    return k(x2, labels2, table)


def kernel(x, sem_labels, embedding_weight, bbox):
    b, t, d_x = x.shape
    n = b * t
    x2 = x.reshape(n, d_x)
    labels2 = sem_labels.reshape(n // _CHUNK, _CHUNK).astype(jnp.int32)
    out2 = _sc_embed_concat(x2, labels2, embedding_weight, n)
    return out2.reshape(b, t, d_x + embedding_weight.shape[1])


# speedup vs baseline: 21.4449x; 21.4449x over previous
import functools

import jax
import jax.numpy as jnp
from jax import lax
from jax.experimental import pallas as pl
from jax.experimental.pallas import tpu as pltpu
from jax.experimental.pallas import tpu_sc as plsc

_NUM_WORKERS = 32
_CHUNK = 128


@functools.partial(jax.jit, static_argnums=(3,))
def _sc_embed_concat(x2, labels2, table, n_tokens):
    d_x = x2.shape[1]
    d_e = table.shape[1]
    d_out = d_x + d_e
    per_w = n_tokens // _NUM_WORKERS
    steps = per_w // _CHUNK
    mesh = plsc.VectorSubcoreMesh(core_axis_name="c", subcore_axis_name="s")

    @functools.partial(
        pl.kernel,
        mesh=mesh,
        out_type=jax.ShapeDtypeStruct((n_tokens, d_out), jnp.float32),
        scratch_types=[
            pltpu.VMEM((steps, _CHUNK), jnp.int32),
            pltpu.VMEM((_CHUNK, d_x), jnp.float32),
            pltpu.VMEM((_CHUNK, d_x), jnp.float32),
            pltpu.VMEM((_CHUNK, d_e), jnp.float32),
            pltpu.VMEM((_CHUNK, d_e), jnp.float32),
        ] + [pltpu.SemaphoreType.DMA] * 8,
    )
    def k(x_hbm, lab_hbm, tab_hbm, out_hbm, idx_v, x0, x1, e0, e1, *sems):
        wid = lax.axis_index("s") * 2 + lax.axis_index("c")
        base = wid * per_w
        xbuf = (x0, x1)
        ebuf = (e0, e1)
        sem_xin = sems[0:2]
        sem_gat = sems[2:4]
        sem_xout = sems[4:6]
        sem_eout = sems[6:8]

        pltpu.sync_copy(lab_hbm.at[pl.ds(wid * steps, steps), :], idx_v)

        def rows(j):
            return pl.ds(base + j * _CHUNK, _CHUNK)

        def start_reads(j):
            p = j % 2
            xin = pltpu.async_copy(x_hbm.at[rows(j), :], xbuf[p], sem_xin[p])
            gat = pltpu.async_copy(tab_hbm.at[idx_v.at[j]], ebuf[p], sem_gat[p])
            return xin, gat

        reads = [None, None]
        writes = [None, None]
        reads[0] = start_reads(0)
        for j in range(steps):
            p = j % 2
            q = (j + 1) % 2
            if j + 1 < steps:
                if writes[q] is not None:
                    writes[q][0].wait()
                    writes[q][1].wait()
                reads[q] = start_reads(j + 1)
            xin, gat = reads[p]
            xin.wait()
            w_x = pltpu.async_copy(xbuf[p], out_hbm.at[rows(j), pl.ds(0, d_x)], sem_xout[p])
            gat.wait()
            w_e = pltpu.async_copy(ebuf[p], out_hbm.at[rows(j), pl.ds(d_x, d_e)], sem_eout[p])
            writes[p] = (w_x, w_e)
        for w in writes:
            w[0].wait()
            w[1].wait()

    return k(x2, labels2, table)


def kernel(x, sem_labels, embedding_weight, bbox):
    b, t, d_x = x.shape
    n = b * t
    x2 = x.reshape(n, d_x)
    labels2 = sem_labels.reshape(n // _CHUNK, _CHUNK).astype(jnp.int32)
    out2 = _sc_embed_concat(x2, labels2, embedding_weight, n)
    return out2.reshape(b, t, d_x + embedding_weight.shape[1])
